# Initial kernel scaffold; baseline (speedup 1.0000x reference)
import jax
import jax.numpy as jnp
from jax.experimental import pallas as pl


def kernel(nm, em, af, edge_index, W, b):
    raise NotImplementedError("write your pallas kernel here")


# speedup vs baseline: 2.6408x; 2.6408x over previous
import functools

import jax
import jax.numpy as jnp
from jax import lax
from jax.experimental import pallas as pl
from jax.experimental.pallas import tpu as pltpu
from jax.experimental.pallas import tpu_sc as plsc

N_NODES = 10000
N_EDGES = 320000
D_NODE = 128
D_EDGE = 16

NC = 2
NS = 16
NW = NC * NS
CH = 64
E_PAD = 327680
EPW = E_PAD // NW
NCHW = EPW // CH
N_PAD = 10240
RPS = N_PAD // NS


def _agg_nm_build():
    mesh = plsc.VectorSubcoreMesh(core_axis_name="c", subcore_axis_name="s")

    @functools.partial(
        pl.kernel,
        mesh=mesh,
        out_type=jax.ShapeDtypeStruct((NC, N_PAD, D_NODE), jnp.float32),
        scratch_types=[
            pltpu.VMEM((2, CH), jnp.int32),
            pltpu.VMEM((2, CH), jnp.int32),
            pltpu.VMEM((CH, D_NODE), jnp.float32),
            pltpu.VMEM((CH, D_NODE), jnp.float32),
            pltpu.VMEM_SHARED((N_PAD, D_NODE), jnp.float32),
            pltpu.SemaphoreType.DMA,
            pltpu.SemaphoreType.DMA,
        ],
    )
    def agg(sd_hbm, nm_hbm, zn_hbm, outn_hbm,
            sd0, sd1, rows0, rows1, acc_n, g0, g1):
        c = lax.axis_index("c")
        s = lax.axis_index("s")
        w = c * NS + s

        pltpu.sync_copy(zn_hbm, acc_n.at[pl.ds(s * RPS, RPS)])
        pltpu.sync_copy(sd_hbm.at[w, 0], sd0)
        pltpu.async_copy(nm_hbm.at[sd0.at[0]], rows0, g0)
        plsc.subcore_barrier()

        def pair(jj, carry):
            j = 2 * jj
            pltpu.sync_copy(sd_hbm.at[w, j + 1], sd1)
            pltpu.async_copy(nm_hbm.at[sd1.at[0]], rows1, g1)
            pltpu.make_async_copy(nm_hbm.at[sd0.at[0]], rows0, g0).wait()
            pltpu.sync_copy(rows0, acc_n.at[sd0.at[1]], add=True)

            @pl.when(jj < NCHW // 2 - 1)
            def _prefetch_even():
                pltpu.sync_copy(sd_hbm.at[w, j + 2], sd0)
                pltpu.async_copy(nm_hbm.at[sd0.at[0]], rows0, g0)

            pltpu.make_async_copy(nm_hbm.at[sd1.at[0]], rows1, g1).wait()
            pltpu.sync_copy(rows1, acc_n.at[sd1.at[1]], add=True)
            return carry

        lax.fori_loop(0, NCHW // 2, pair, 0)
        plsc.subcore_barrier()

        pltpu.sync_copy(acc_n.at[pl.ds(s * RPS, RPS)],
                        outn_hbm.at[c, pl.ds(s * RPS, RPS)])

    return agg


def _agg_em_build():
    mesh = plsc.VectorSubcoreMesh(core_axis_name="c", subcore_axis_name="s")

    @functools.partial(
        pl.kernel,
        mesh=mesh,
        out_type=jax.ShapeDtypeStruct((NC, N_PAD, D_NODE), jnp.float32),
        scratch_types=[
            pltpu.VMEM((2, CH), jnp.int32),
            pltpu.VMEM((D_EDGE, CH), jnp.float32),
            pltpu.VMEM((CH, D_NODE), jnp.float32),
            pltpu.VMEM_SHARED((N_PAD, D_NODE), jnp.float32),
        ],
    )
    def agg(sd_hbm, em_hbm, zn_hbm, oute_hbm, sd, em16, em_wide, acc_e):
        c = lax.axis_index("c")
        s = lax.axis_index("s")
        w = c * NS + s

        pltpu.sync_copy(zn_hbm, acc_e.at[pl.ds(s * RPS, RPS)])

        zvec = jnp.zeros((16,), jnp.float32)

        def zw_body(i, carry):
            em_wide[i // 8, pl.ds((i % 8) * 16, 16)] = zvec
            return carry

        lax.fori_loop(0, CH * 8, zw_body, 0)
        plsc.subcore_barrier()

        def chunk(j, carry):
            pltpu.sync_copy(sd_hbm.at[w, j], sd)
            pltpu.sync_copy(em_hbm.at[w, j], em16)
            for r in range(D_EDGE):
                for q in range(CH // 16):
                    em_wide[r * (CH // 16) + q, pl.ds(0, 16)] = (
                        em16[r, pl.ds(q * 16, 16)])
            pltpu.sync_copy(em_wide, acc_e.at[sd.at[1]], add=True)
            return carry

        lax.fori_loop(0, NCHW, chunk, 0)
        plsc.subcore_barrier()

        pltpu.sync_copy(acc_e.at[pl.ds(s * RPS, RPS)],
                        oute_hbm.at[c, pl.ds(s * RPS, RPS)])

    return agg


_agg_nm = _agg_nm_build()
_agg_em = _agg_em_build()

BT = 1000


def _node_update_body(sn_ref, se_ref, af_ref, wnT_ref, weT_ref, b_ref, o_ref):
    sn = sn_ref[0] + sn_ref[1]
    se = se_ref[0] + se_ref[1]
    y = jnp.dot(sn, wnT_ref[...], preferred_element_type=jnp.float32)
    y = y + jnp.dot(se, weT_ref[...], preferred_element_type=jnp.float32)
    y = y + af_ref[...] + b_ref[...]
    o_ref[...] = jnp.where(y >= 0, y, 0.1 * y)


def _node_update(sn, se, af, wnT, weT, b2):
    return pl.pallas_call(
        _node_update_body,
        grid=(N_NODES // BT,),
        in_specs=[
            pl.BlockSpec((NC, BT, D_NODE), lambda i: (0, i, 0)),
            pl.BlockSpec((NC, BT, D_NODE), lambda i: (0, i, 0)),
            pl.BlockSpec((BT, D_NODE), lambda i: (i, 0)),
            pl.BlockSpec((D_NODE, D_NODE), lambda i: (0, 0)),
            pl.BlockSpec((D_NODE, D_NODE), lambda i: (0, 0)),
            pl.BlockSpec((1, D_NODE), lambda i: (0, 0)),
        ],
        out_specs=pl.BlockSpec((BT, D_NODE), lambda i: (i, 0)),
        out_shape=jax.ShapeDtypeStruct((N_NODES, D_NODE), jnp.float32),
    )(sn, se, af, wnT, weT, b2)


def kernel(nm, em, af, edge_index, W, b):
    ei = edge_index.astype(jnp.int32)
    pad = E_PAD - N_EDGES
    src_p = jnp.concatenate([ei[0], jnp.zeros((pad,), jnp.int32)])
    dst_p = jnp.concatenate([ei[1], jnp.full((pad,), N_NODES, jnp.int32)])
    sd = jnp.stack([src_p.reshape(NW, NCHW, CH),
                    dst_p.reshape(NW, NCHW, CH)], axis=2)
    em_p = jnp.concatenate([em, jnp.zeros((pad, D_EDGE), jnp.float32)])
    em4 = em_p.reshape(NW, NCHW, D_EDGE, CH)
    zn = jnp.zeros((RPS, D_NODE), jnp.float32)
    sn = _agg_nm(sd, nm, zn)
    se = _agg_em(sd, em4, zn)
    wnT = W[:, :D_NODE].T
    weT = jnp.zeros((D_NODE, D_NODE), jnp.float32).at[:D_EDGE].set(W[:, D_NODE:].T)
    b2 = b.reshape(1, D_NODE)
    return _node_update(sn, se, af, wnT, weT, b2)


# speedup vs baseline: 3.0240x; 1.1451x over previous
import functools

import jax
import jax.numpy as jnp
from jax import lax
from jax.experimental import pallas as pl
from jax.experimental.pallas import tpu as pltpu
from jax.experimental.pallas import tpu_sc as plsc

N_NODES = 10000
N_EDGES = 320000
D_NODE = 128
D_EDGE = 16

NC = 2
NS = 16
NW = NC * NS
CH = 128
E_PAD = 327680
EPW = E_PAD // NW
NCHW = EPW // CH
N_PAD = 10240
RPS = N_PAD // NS


def _agg_nm_build():
    mesh = plsc.VectorSubcoreMesh(core_axis_name="c", subcore_axis_name="s")

    @functools.partial(
        pl.kernel,
        mesh=mesh,
        out_type=jax.ShapeDtypeStruct((NC, N_PAD, D_NODE), jnp.float32),
        scratch_types=[
            pltpu.VMEM((2, CH), jnp.int32),
            pltpu.VMEM((2, CH), jnp.int32),
            pltpu.VMEM((CH, D_NODE), jnp.float32),
            pltpu.VMEM((CH, D_NODE), jnp.float32),
            pltpu.VMEM_SHARED((N_PAD, D_NODE), jnp.float32),
            pltpu.SemaphoreType.DMA,
            pltpu.SemaphoreType.DMA,
        ],
    )
    def agg(sd_hbm, nm_hbm, zn_hbm, outn_hbm,
            sd0, sd1, rows0, rows1, acc_n, g0, g1):
        c = lax.axis_index("c")
        s = lax.axis_index("s")
        w = c * NS + s

        pltpu.sync_copy(zn_hbm, acc_n.at[pl.ds(s * RPS, RPS)])
        pltpu.sync_copy(sd_hbm.at[w, 0], sd0)
        pltpu.async_copy(nm_hbm.at[sd0.at[0]], rows0, g0)
        plsc.subcore_barrier()

        def pair(jj, carry):
            j = 2 * jj
            pltpu.sync_copy(sd_hbm.at[w, j + 1], sd1)
            pltpu.async_copy(nm_hbm.at[sd1.at[0]], rows1, g1)
            pltpu.make_async_copy(nm_hbm.at[sd0.at[0]], rows0, g0).wait()
            pltpu.sync_copy(rows0, acc_n.at[sd0.at[1]], add=True)

            @pl.when(jj < NCHW // 2 - 1)
            def _prefetch_even():
                pltpu.sync_copy(sd_hbm.at[w, j + 2], sd0)
                pltpu.async_copy(nm_hbm.at[sd0.at[0]], rows0, g0)

            pltpu.make_async_copy(nm_hbm.at[sd1.at[0]], rows1, g1).wait()
            pltpu.sync_copy(rows1, acc_n.at[sd1.at[1]], add=True)
            return carry

        lax.fori_loop(0, NCHW // 2, pair, 0)
        plsc.subcore_barrier()

        pltpu.sync_copy(acc_n.at[pl.ds(s * RPS, RPS)],
                        outn_hbm.at[c, pl.ds(s * RPS, RPS)])

    return agg


def _agg_em_build():
    mesh = plsc.VectorSubcoreMesh(core_axis_name="c", subcore_axis_name="s")

    @functools.partial(
        pl.kernel,
        mesh=mesh,
        out_type=jax.ShapeDtypeStruct((NC, N_PAD, D_NODE), jnp.float32),
        scratch_types=[
            pltpu.VMEM((2, CH), jnp.int32),
            pltpu.VMEM((D_EDGE, CH), jnp.float32),
            pltpu.VMEM((CH, D_NODE), jnp.float32),
            pltpu.VMEM_SHARED((N_PAD, D_NODE), jnp.float32),
        ],
    )
    def agg(sd_hbm, em_hbm, zn_hbm, oute_hbm, sd, em16, em_wide, acc_e):
        c = lax.axis_index("c")
        s = lax.axis_index("s")
        w = c * NS + s

        pltpu.sync_copy(zn_hbm, acc_e.at[pl.ds(s * RPS, RPS)])

        zvec = jnp.zeros((16,), jnp.float32)

        def zw_body(i, carry):
            em_wide[i // 8, pl.ds((i % 8) * 16, 16)] = zvec
            return carry

        lax.fori_loop(0, CH * 8, zw_body, 0)
        plsc.subcore_barrier()

        def chunk(j, carry):
            pltpu.sync_copy(sd_hbm.at[w, j], sd)
            pltpu.sync_copy(em_hbm.at[w, j], em16)
            for r in range(D_EDGE):
                for q in range(CH // 16):
                    em_wide[r * (CH // 16) + q, pl.ds(0, 16)] = (
                        em16[r, pl.ds(q * 16, 16)])
            pltpu.sync_copy(em_wide, acc_e.at[sd.at[1]], add=True)
            return carry

        lax.fori_loop(0, NCHW, chunk, 0)
        plsc.subcore_barrier()

        pltpu.sync_copy(acc_e.at[pl.ds(s * RPS, RPS)],
                        oute_hbm.at[c, pl.ds(s * RPS, RPS)])

    return agg


_agg_nm = _agg_nm_build()
_agg_em = _agg_em_build()

BT = 1000


def _node_update_body(sn_ref, se_ref, af_ref, wnT_ref, weT_ref, b_ref, o_ref):
    sn = sn_ref[0] + sn_ref[1]
    se = se_ref[0] + se_ref[1]
    y = jnp.dot(sn, wnT_ref[...], preferred_element_type=jnp.float32)
    y = y + jnp.dot(se, weT_ref[...], preferred_element_type=jnp.float32)
    y = y + af_ref[...] + b_ref[...]
    o_ref[...] = jnp.where(y >= 0, y, 0.1 * y)


def _node_update(sn, se, af, wnT, weT, b2):
    return pl.pallas_call(
        _node_update_body,
        grid=(N_NODES // BT,),
        in_specs=[
            pl.BlockSpec((NC, BT, D_NODE), lambda i: (0, i, 0)),
            pl.BlockSpec((NC, BT, D_NODE), lambda i: (0, i, 0)),
            pl.BlockSpec((BT, D_NODE), lambda i: (i, 0)),
            pl.BlockSpec((D_NODE, D_NODE), lambda i: (0, 0)),
            pl.BlockSpec((D_NODE, D_NODE), lambda i: (0, 0)),
            pl.BlockSpec((1, D_NODE), lambda i: (0, 0)),
        ],
        out_specs=pl.BlockSpec((BT, D_NODE), lambda i: (i, 0)),
        out_shape=jax.ShapeDtypeStruct((N_NODES, D_NODE), jnp.float32),
    )(sn, se, af, wnT, weT, b2)


def kernel(nm, em, af, edge_index, W, b):
    ei = edge_index.astype(jnp.int32)
    pad = E_PAD - N_EDGES
    src_p = jnp.concatenate([ei[0], jnp.zeros((pad,), jnp.int32)])
    dst_p = jnp.concatenate([ei[1], jnp.full((pad,), N_NODES, jnp.int32)])
    sd = jnp.stack([src_p.reshape(NW, NCHW, CH),
                    dst_p.reshape(NW, NCHW, CH)], axis=2)
    em_p = jnp.concatenate([em, jnp.zeros((pad, D_EDGE), jnp.float32)])
    em4 = em_p.reshape(NW, NCHW, D_EDGE, CH)
    zn = jnp.zeros((RPS, D_NODE), jnp.float32)
    sn = _agg_nm(sd, nm, zn)
    se = _agg_em(sd, em4, zn)
    wnT = W[:, :D_NODE].T
    weT = jnp.zeros((D_NODE, D_NODE), jnp.float32).at[:D_EDGE].set(W[:, D_NODE:].T)
    b2 = b.reshape(1, D_NODE)
    return _node_update(sn, se, af, wnT, weT, b2)


# speedup vs baseline: 3.2285x; 1.0676x over previous
import functools

import jax
import jax.numpy as jnp
from jax import lax
from jax.experimental import pallas as pl
from jax.experimental.pallas import tpu as pltpu
from jax.experimental.pallas import tpu_sc as plsc

N_NODES = 10000
N_EDGES = 320000
D_NODE = 128
D_EDGE = 16

NC = 2
NS = 16
NW = NC * NS
CH = 128
E_PAD = 327680
EPW = E_PAD // NW
NCHW = EPW // CH
N_PAD = 10240
RPS = N_PAD // NS


def _agg_nm_build():
    mesh = plsc.VectorSubcoreMesh(core_axis_name="c", subcore_axis_name="s")

    @functools.partial(
        pl.kernel,
        mesh=mesh,
        out_type=jax.ShapeDtypeStruct((NC, N_PAD, D_NODE), jnp.float32),
        scratch_types=[
            pltpu.VMEM((2, CH), jnp.int32),
            pltpu.VMEM((2, CH), jnp.int32),
            pltpu.VMEM((CH, D_NODE), jnp.float32),
            pltpu.VMEM((CH, D_NODE), jnp.float32),
            pltpu.VMEM_SHARED((N_PAD, D_NODE), jnp.float32),
            pltpu.SemaphoreType.DMA,
            pltpu.SemaphoreType.DMA,
        ],
    )
    def agg(sd_hbm, nm_hbm, zn_hbm, outn_hbm,
            sd0, sd1, rows0, rows1, acc_n, g0, g1):
        c = lax.axis_index("c")
        s = lax.axis_index("s")
        w = c * NS + s

        pltpu.sync_copy(zn_hbm, acc_n.at[pl.ds(s * RPS, RPS)])
        pltpu.sync_copy(sd_hbm.at[w, 0], sd0)
        pltpu.async_copy(nm_hbm.at[sd0.at[0]], rows0, g0)
        plsc.subcore_barrier()

        def pair(jj, carry):
            j = 2 * jj
            pltpu.sync_copy(sd_hbm.at[w, j + 1], sd1)
            pltpu.async_copy(nm_hbm.at[sd1.at[0]], rows1, g1)
            pltpu.make_async_copy(nm_hbm.at[sd0.at[0]], rows0, g0).wait()
            pltpu.sync_copy(rows0, acc_n.at[sd0.at[1]], add=True)

            @pl.when(jj < NCHW // 2 - 1)
            def _prefetch_even():
                pltpu.sync_copy(sd_hbm.at[w, j + 2], sd0)
                pltpu.async_copy(nm_hbm.at[sd0.at[0]], rows0, g0)

            pltpu.make_async_copy(nm_hbm.at[sd1.at[0]], rows1, g1).wait()
            pltpu.sync_copy(rows1, acc_n.at[sd1.at[1]], add=True)
            return carry

        lax.fori_loop(0, NCHW // 2, pair, 0)
        plsc.subcore_barrier()

        pltpu.sync_copy(acc_n.at[pl.ds(s * RPS, RPS)],
                        outn_hbm.at[c, pl.ds(s * RPS, RPS)])

    return agg


def _agg_em_build():
    mesh = plsc.VectorSubcoreMesh(core_axis_name="c", subcore_axis_name="s")

    @functools.partial(
        pl.kernel,
        mesh=mesh,
        out_type=jax.ShapeDtypeStruct((NC, N_PAD, D_NODE), jnp.float32),
        scratch_types=[
            pltpu.VMEM((2, CH), jnp.int32),
            pltpu.VMEM((2, CH), jnp.int32),
            pltpu.VMEM((D_EDGE, CH), jnp.float32),
            pltpu.VMEM((D_EDGE, CH), jnp.float32),
            pltpu.VMEM((CH, D_NODE), jnp.float32),
            pltpu.VMEM((CH, D_NODE), jnp.float32),
            pltpu.VMEM_SHARED((N_PAD, D_NODE), jnp.float32),
            pltpu.SemaphoreType.DMA,
            pltpu.SemaphoreType.DMA,
        ],
    )
    def agg(sd_hbm, em_hbm, zn_hbm, oute_hbm,
            sd0, sd1, em0, em1, wide0, wide1, acc_e, t0, t1):
        c = lax.axis_index("c")
        s = lax.axis_index("s")
        w = c * NS + s

        pltpu.sync_copy(zn_hbm, acc_e.at[pl.ds(s * RPS, RPS)])

        zvec = jnp.zeros((16,), jnp.float32)

        def zw_body(i, carry):
            wide0[i // 8, pl.ds((i % 8) * 16, 16)] = zvec
            wide1[i // 8, pl.ds((i % 8) * 16, 16)] = zvec
            return carry

        lax.fori_loop(0, CH * 8, zw_body, 0)
        plsc.subcore_barrier()

        def spread(em16, wide):
            for r in range(D_EDGE):
                for q in range(CH // 16):
                    wide[r * (CH // 16) + q, pl.ds(0, 16)] = (
                        em16[r, pl.ds(q * 16, 16)])

        def pair(jj, carry):
            j = 2 * jj
            @pl.when(jj > 0)
            def _drain_t0():
                pltpu.make_async_copy(wide0, acc_e.at[sd0.at[1]], t0).wait()

            pltpu.sync_copy(sd_hbm.at[w, j], sd0)
            pltpu.sync_copy(em_hbm.at[w, j], em0)
            spread(em0, wide0)
            pltpu.async_copy(wide0, acc_e.at[sd0.at[1]], t0, add=True)

            @pl.when(jj > 0)
            def _drain_t1():
                pltpu.make_async_copy(wide1, acc_e.at[sd1.at[1]], t1).wait()

            pltpu.sync_copy(sd_hbm.at[w, j + 1], sd1)
            pltpu.sync_copy(em_hbm.at[w, j + 1], em1)
            spread(em1, wide1)
            pltpu.async_copy(wide1, acc_e.at[sd1.at[1]], t1, add=True)
            return carry

        lax.fori_loop(0, NCHW // 2, pair, 0)
        pltpu.make_async_copy(wide0, acc_e.at[sd0.at[1]], t0).wait()
        pltpu.make_async_copy(wide1, acc_e.at[sd1.at[1]], t1).wait()
        plsc.subcore_barrier()

        pltpu.sync_copy(acc_e.at[pl.ds(s * RPS, RPS)],
                        oute_hbm.at[c, pl.ds(s * RPS, RPS)])

    return agg


_agg_nm = _agg_nm_build()
_agg_em = _agg_em_build()

BT = 1000


def _node_update_body(sn_ref, se_ref, af_ref, wnT_ref, weT_ref, b_ref, o_ref):
    sn = sn_ref[0] + sn_ref[1]
    se = se_ref[0] + se_ref[1]
    y = jnp.dot(sn, wnT_ref[...], preferred_element_type=jnp.float32)
    y = y + jnp.dot(se, weT_ref[...], preferred_element_type=jnp.float32)
    y = y + af_ref[...] + b_ref[...]
    o_ref[...] = jnp.where(y >= 0, y, 0.1 * y)


def _node_update(sn, se, af, wnT, weT, b2):
    return pl.pallas_call(
        _node_update_body,
        grid=(N_NODES // BT,),
        in_specs=[
            pl.BlockSpec((NC, BT, D_NODE), lambda i: (0, i, 0)),
            pl.BlockSpec((NC, BT, D_NODE), lambda i: (0, i, 0)),
            pl.BlockSpec((BT, D_NODE), lambda i: (i, 0)),
            pl.BlockSpec((D_NODE, D_NODE), lambda i: (0, 0)),
            pl.BlockSpec((D_NODE, D_NODE), lambda i: (0, 0)),
            pl.BlockSpec((1, D_NODE), lambda i: (0, 0)),
        ],
        out_specs=pl.BlockSpec((BT, D_NODE), lambda i: (i, 0)),
        out_shape=jax.ShapeDtypeStruct((N_NODES, D_NODE), jnp.float32),
    )(sn, se, af, wnT, weT, b2)


def kernel(nm, em, af, edge_index, W, b):
    ei = edge_index.astype(jnp.int32)
    pad = E_PAD - N_EDGES
    src_p = jnp.concatenate([ei[0], jnp.zeros((pad,), jnp.int32)])
    dst_p = jnp.concatenate([ei[1], jnp.full((pad,), N_NODES, jnp.int32)])
    sd = jnp.stack([src_p.reshape(NW, NCHW, CH),
                    dst_p.reshape(NW, NCHW, CH)], axis=2)
    em_p = jnp.concatenate([em, jnp.zeros((pad, D_EDGE), jnp.float32)])
    em4 = em_p.reshape(NW, NCHW, D_EDGE, CH)
    zn = jnp.zeros((RPS, D_NODE), jnp.float32)
    sn = _agg_nm(sd, nm, zn)
    se = _agg_em(sd, em4, zn)
    wnT = W[:, :D_NODE].T
    weT = jnp.zeros((D_NODE, D_NODE), jnp.float32).at[:D_EDGE].set(W[:, D_NODE:].T)
    b2 = b.reshape(1, D_NODE)
    return _node_update(sn, se, af, wnT, weT, b2)


# speedup vs baseline: 3.2413x; 1.0040x over previous
import functools

import jax
import jax.numpy as jnp
from jax import lax
from jax.experimental import pallas as pl
from jax.experimental.pallas import tpu as pltpu
from jax.experimental.pallas import tpu_sc as plsc

N_NODES = 10000
N_EDGES = 320000
D_NODE = 128
D_EDGE = 16

NC = 2
NS = 16
NW = NC * NS
CH = 128
E_PAD = 327680
EPW = E_PAD // NW
NCHW = EPW // CH
N_PAD = 10240
RPS = N_PAD // NS


def _agg_nm_build():
    mesh = plsc.VectorSubcoreMesh(core_axis_name="c", subcore_axis_name="s")

    SDH = NCHW // 2

    @functools.partial(
        pl.kernel,
        mesh=mesh,
        out_type=jax.ShapeDtypeStruct((NC, N_PAD, D_NODE), jnp.float32),
        scratch_types=[
            pltpu.VMEM((SDH, 2, CH), jnp.int32),
            pltpu.VMEM((CH, D_NODE), jnp.float32),
            pltpu.VMEM((CH, D_NODE), jnp.float32),
            pltpu.VMEM_SHARED((N_PAD, D_NODE), jnp.float32),
            pltpu.SemaphoreType.DMA,
            pltpu.SemaphoreType.DMA,
            pltpu.SemaphoreType.DMA,
            pltpu.SemaphoreType.DMA,
        ],
    )
    def agg(sd_hbm, nm_hbm, zn_hbm, outn_hbm,
            sd_all, rows0, rows1, acc_n, g0, g1, t0, t1):
        c = lax.axis_index("c")
        s = lax.axis_index("s")
        w = c * NS + s

        pltpu.sync_copy(zn_hbm, acc_n.at[pl.ds(s * RPS, RPS)])
        plsc.subcore_barrier()

        for h in range(2):
            pltpu.sync_copy(sd_hbm.at[w, pl.ds(h * SDH, SDH)], sd_all)
            pltpu.async_copy(nm_hbm.at[sd_all.at[0, 0]], rows0, g0)

            def pair(jj, carry):
                a = 2 * jj

                @pl.when(jj > 0)
                def _drain_t1():
                    pltpu.make_async_copy(
                        rows1, acc_n.at[sd_all.at[0, 1]], t1).wait()

                pltpu.async_copy(nm_hbm.at[sd_all.at[a + 1, 0]], rows1, g1)
                pltpu.make_async_copy(
                    nm_hbm.at[sd_all.at[a, 0]], rows0, g0).wait()
                pltpu.async_copy(rows0, acc_n.at[sd_all.at[a, 1]], t0, add=True)

                @pl.when(jj < SDH // 2 - 1)
                def _prefetch_even():
                    pltpu.make_async_copy(
                        rows0, acc_n.at[sd_all.at[0, 1]], t0).wait()
                    pltpu.async_copy(nm_hbm.at[sd_all.at[a + 2, 0]], rows0, g0)

                pltpu.make_async_copy(
                    nm_hbm.at[sd_all.at[a + 1, 0]], rows1, g1).wait()
                pltpu.async_copy(rows1, acc_n.at[sd_all.at[a + 1, 1]], t1,
                                 add=True)
                return carry

            lax.fori_loop(0, SDH // 2, pair, 0)
            pltpu.make_async_copy(rows0, acc_n.at[sd_all.at[0, 1]], t0).wait()
            pltpu.make_async_copy(rows1, acc_n.at[sd_all.at[0, 1]], t1).wait()

        plsc.subcore_barrier()

        pltpu.sync_copy(acc_n.at[pl.ds(s * RPS, RPS)],
                        outn_hbm.at[c, pl.ds(s * RPS, RPS)])

    return agg


def _agg_em_build():
    mesh = plsc.VectorSubcoreMesh(core_axis_name="c", subcore_axis_name="s")

    @functools.partial(
        pl.kernel,
        mesh=mesh,
        out_type=jax.ShapeDtypeStruct((NC, N_PAD, D_NODE), jnp.float32),
        scratch_types=[
            pltpu.VMEM((2, CH), jnp.int32),
            pltpu.VMEM((2, CH), jnp.int32),
            pltpu.VMEM((D_EDGE, CH), jnp.float32),
            pltpu.VMEM((D_EDGE, CH), jnp.float32),
            pltpu.VMEM((CH, D_NODE), jnp.float32),
            pltpu.VMEM((CH, D_NODE), jnp.float32),
            pltpu.VMEM_SHARED((N_PAD, D_NODE), jnp.float32),
            pltpu.SemaphoreType.DMA,
            pltpu.SemaphoreType.DMA,
        ],
    )
    def agg(sd_hbm, em_hbm, zn_hbm, oute_hbm,
            sd0, sd1, em0, em1, wide0, wide1, acc_e, t0, t1):
        c = lax.axis_index("c")
        s = lax.axis_index("s")
        w = c * NS + s

        pltpu.sync_copy(zn_hbm, acc_e.at[pl.ds(s * RPS, RPS)])

        zvec = jnp.zeros((16,), jnp.float32)

        def zw_body(i, carry):
            wide0[i // 8, pl.ds((i % 8) * 16, 16)] = zvec
            wide1[i // 8, pl.ds((i % 8) * 16, 16)] = zvec
            return carry

        lax.fori_loop(0, CH * 8, zw_body, 0)
        plsc.subcore_barrier()

        def spread(em16, wide):
            for r in range(D_EDGE):
                for q in range(CH // 16):
                    wide[r * (CH // 16) + q, pl.ds(0, 16)] = (
                        em16[r, pl.ds(q * 16, 16)])

        def pair(jj, carry):
            j = 2 * jj
            @pl.when(jj > 0)
            def _drain_t0():
                pltpu.make_async_copy(wide0, acc_e.at[sd0.at[1]], t0).wait()

            pltpu.sync_copy(sd_hbm.at[w, j], sd0)
            pltpu.sync_copy(em_hbm.at[w, j], em0)
            spread(em0, wide0)
            pltpu.async_copy(wide0, acc_e.at[sd0.at[1]], t0, add=True)

            @pl.when(jj > 0)
            def _drain_t1():
                pltpu.make_async_copy(wide1, acc_e.at[sd1.at[1]], t1).wait()

            pltpu.sync_copy(sd_hbm.at[w, j + 1], sd1)
            pltpu.sync_copy(em_hbm.at[w, j + 1], em1)
            spread(em1, wide1)
            pltpu.async_copy(wide1, acc_e.at[sd1.at[1]], t1, add=True)
            return carry

        lax.fori_loop(0, NCHW // 2, pair, 0)
        pltpu.make_async_copy(wide0, acc_e.at[sd0.at[1]], t0).wait()
        pltpu.make_async_copy(wide1, acc_e.at[sd1.at[1]], t1).wait()
        plsc.subcore_barrier()

        pltpu.sync_copy(acc_e.at[pl.ds(s * RPS, RPS)],
                        oute_hbm.at[c, pl.ds(s * RPS, RPS)])

    return agg


_agg_nm = _agg_nm_build()
_agg_em = _agg_em_build()

BT = 1000


def _node_update_body(sn_ref, se_ref, af_ref, wnT_ref, weT_ref, b_ref, o_ref):
    sn = sn_ref[0] + sn_ref[1]
    se = se_ref[0] + se_ref[1]
    y = jnp.dot(sn, wnT_ref[...], preferred_element_type=jnp.float32)
    y = y + jnp.dot(se, weT_ref[...], preferred_element_type=jnp.float32)
    y = y + af_ref[...] + b_ref[...]
    o_ref[...] = jnp.where(y >= 0, y, 0.1 * y)


def _node_update(sn, se, af, wnT, weT, b2):
    return pl.pallas_call(
        _node_update_body,
        grid=(N_NODES // BT,),
        in_specs=[
            pl.BlockSpec((NC, BT, D_NODE), lambda i: (0, i, 0)),
            pl.BlockSpec((NC, BT, D_NODE), lambda i: (0, i, 0)),
            pl.BlockSpec((BT, D_NODE), lambda i: (i, 0)),
            pl.BlockSpec((D_NODE, D_NODE), lambda i: (0, 0)),
            pl.BlockSpec((D_NODE, D_NODE), lambda i: (0, 0)),
            pl.BlockSpec((1, D_NODE), lambda i: (0, 0)),
        ],
        out_specs=pl.BlockSpec((BT, D_NODE), lambda i: (i, 0)),
        out_shape=jax.ShapeDtypeStruct((N_NODES, D_NODE), jnp.float32),
    )(sn, se, af, wnT, weT, b2)


def kernel(nm, em, af, edge_index, W, b):
    ei = edge_index.astype(jnp.int32)
    pad = E_PAD - N_EDGES
    src_p = jnp.concatenate([ei[0], jnp.zeros((pad,), jnp.int32)])
    dst_p = jnp.concatenate([ei[1], jnp.full((pad,), N_NODES, jnp.int32)])
    sd = jnp.stack([src_p.reshape(NW, NCHW, CH),
                    dst_p.reshape(NW, NCHW, CH)], axis=2)
    em_p = jnp.concatenate([em, jnp.zeros((pad, D_EDGE), jnp.float32)])
    em4 = em_p.reshape(NW, NCHW, D_EDGE, CH)
    zn = jnp.zeros((RPS, D_NODE), jnp.float32)
    sn = _agg_nm(sd, nm, zn)
    se = _agg_em(sd, em4, zn)
    wnT = W[:, :D_NODE].T
    weT = jnp.zeros((D_NODE, D_NODE), jnp.float32).at[:D_EDGE].set(W[:, D_NODE:].T)
    b2 = b.reshape(1, D_NODE)
    return _node_update(sn, se, af, wnT, weT, b2)
